# baseline (speedup 1.0000x reference)
import jax
import jax.numpy as jnp
from jax import lax
from jax.experimental import pallas as pl
from jax.experimental.pallas import tpu as pltpu
from jax.experimental.pallas import tpu_sc as plsc

NUM_CODES = 1024
DIM = 64
PIX = 1024

_NC = 2
_NS = 16
_NW = _NC * _NS


def _argmin_body(x_ref, cb_ref, zis_ref, cbn2_ref, c2_ref):
    @pl.when(pl.program_id(0) == 0)
    def _init():
        cb0 = cb_ref[...]
        cbn2_ref[...] = cb0 * -2.0
        c2_ref[...] = jnp.sum(cb0 * cb0, axis=1, keepdims=True)

    x = x_ref[...]

    mmn2 = lax.dot_general(cbn2_ref[...], x, (((1,), (0,)), ((), ())),
                           precision=lax.Precision.DEFAULT)
    z2 = jnp.sum(x * x, axis=0)
    dist = (z2[None, :] + c2_ref[...]) + mmn2

    zis_ref[...] = jnp.argmin(dist, axis=0).reshape(8, 128)


def _tc_argmin(x, codebook):
    B = x.shape[0]
    return pl.pallas_call(
        _argmin_body,
        grid=(B,),
        in_specs=[
            pl.BlockSpec((None, DIM, PIX), lambda b: (b, 0, 0)),
            pl.BlockSpec((NUM_CODES, DIM), lambda b: (0, 0)),
        ],
        out_specs=pl.BlockSpec((None, 8, 128), lambda b: (b, 0, 0)),
        out_shape=jax.ShapeDtypeStruct((B, 8, 128), jnp.int32),
        scratch_shapes=[
            pltpu.VMEM((NUM_CODES, DIM), jnp.float32),
            pltpu.VMEM((NUM_CODES, 1), jnp.float32),
        ],
    )(x, codebook)


def _sc_lookup_body(cb_hbm, zis_hbm, out_hbm, idx_v, rows_v, sem):
    n_tok = idx_v.shape[0]
    wid = lax.axis_index("s") * _NC + lax.axis_index("c")
    base = wid * n_tok
    pltpu.sync_copy(zis_hbm.at[pl.ds(base, n_tok)], idx_v)
    pltpu.async_copy(cb_hbm.at[idx_v], rows_v, sem).wait()
    pltpu.sync_copy(rows_v, out_hbm.at[pl.ds(base, n_tok)])


def _sc_lookup(cb_pad, zis_flat, n):
    n_tok = n // _NW
    mesh = plsc.VectorSubcoreMesh(core_axis_name="c", subcore_axis_name="s")
    f = pl.kernel(
        _sc_lookup_body,
        out_type=jax.ShapeDtypeStruct((n, 128), jnp.float32),
        mesh=mesh,
        scratch_types=[
            pltpu.VMEM((n_tok,), jnp.int32),
            pltpu.VMEM((n_tok, 128), jnp.float32),
            pltpu.SemaphoreType.DMA,
        ],
        compiler_params=pltpu.CompilerParams(needs_layout_passes=False),
    )
    return f(cb_pad, zis_flat)


def kernel(inputs, codebook):
    B = inputs.shape[0]
    x = inputs.reshape(B, DIM, PIX)
    zis = _tc_argmin(x, codebook)
    cb_pad = jnp.concatenate(
        [codebook, jnp.zeros((NUM_CODES, 128 - DIM), jnp.float32)], axis=1)
    rows = _sc_lookup(cb_pad, zis.reshape(B * PIX), B * PIX)
    zqs = rows[:, :DIM].reshape(B, PIX, DIM).transpose(0, 2, 1)
    return (zis.reshape(B, 32, 32), zqs.reshape(B, DIM, 32, 32))


# speedup vs baseline: 1.2665x; 1.2665x over previous
import jax
import jax.numpy as jnp
from jax import lax
from jax.experimental import pallas as pl
from jax.experimental.pallas import tpu as pltpu
from jax.experimental.pallas import tpu_sc as plsc

NUM_CODES = 1024
DIM = 64
PIX = 1024

_NC = 2
_NS = 16
_NW = _NC * _NS


def _argmin_body(x_ref, cb_ref, zis_ref, cbn2_ref, c2_ref):
    @pl.when(pl.program_id(0) == 0)
    def _init():
        cb0 = cb_ref[...]
        cbn2_ref[...] = cb0 * -2.0
        c2_ref[...] = jnp.sum(cb0 * cb0, axis=1, keepdims=True)

    x = x_ref[...]

    mmn2 = lax.dot_general(cbn2_ref[...], x, (((1,), (0,)), ((), ())),
                           precision=lax.Precision.DEFAULT)
    z2 = jnp.sum(x * x, axis=0)
    dist = (z2[None, :] + c2_ref[...]) + mmn2

    zis_ref[...] = jnp.argmin(dist, axis=0).reshape(8, 128)


def _tc_argmin(x, codebook):
    B = x.shape[0]
    return pl.pallas_call(
        _argmin_body,
        grid=(B,),
        in_specs=[
            pl.BlockSpec((None, DIM, PIX), lambda b: (b, 0, 0)),
            pl.BlockSpec((NUM_CODES, DIM), lambda b: (0, 0)),
        ],
        out_specs=pl.BlockSpec((None, 8, 128), lambda b: (b, 0, 0)),
        out_shape=jax.ShapeDtypeStruct((B, 8, 128), jnp.int32),
        scratch_shapes=[
            pltpu.VMEM((NUM_CODES, DIM), jnp.float32),
            pltpu.VMEM((NUM_CODES, 1), jnp.float32),
        ],
    )(x, codebook)


def _sc_lookup_body(cb_hbm, zis_hbm, out_hbm, cb_sh, idx_v, rows_v, sem):
    n_tok = idx_v.shape[0]
    sid = lax.axis_index("s")
    wid = sid * _NC + lax.axis_index("c")
    base = wid * n_tok

    @pl.when(sid == 0)
    def _stage():
        pltpu.sync_copy(cb_hbm, cb_sh)

    pltpu.sync_copy(zis_hbm.at[pl.ds(base, n_tok)], idx_v)
    plsc.subcore_barrier()
    pltpu.async_copy(cb_sh.at[idx_v], rows_v, sem).wait()
    pltpu.sync_copy(rows_v, out_hbm.at[pl.ds(base, n_tok)])


def _sc_lookup(cb_pad, zis_flat, n):
    n_tok = n // _NW
    mesh = plsc.VectorSubcoreMesh(core_axis_name="c", subcore_axis_name="s")
    f = pl.kernel(
        _sc_lookup_body,
        out_type=jax.ShapeDtypeStruct((n, 128), jnp.float32),
        mesh=mesh,
        scratch_types=[
            pltpu.VMEM_SHARED((NUM_CODES, 128), jnp.float32),
            pltpu.VMEM((n_tok,), jnp.int32),
            pltpu.VMEM((n_tok, 128), jnp.float32),
            pltpu.SemaphoreType.DMA,
        ],
        compiler_params=pltpu.CompilerParams(needs_layout_passes=False),
    )
    return f(cb_pad, zis_flat)


def kernel(inputs, codebook):
    B = inputs.shape[0]
    x = inputs.reshape(B, DIM, PIX)
    zis = _tc_argmin(x, codebook)
    cb_pad = jnp.concatenate(
        [codebook, jnp.zeros((NUM_CODES, 128 - DIM), jnp.float32)], axis=1)
    rows = _sc_lookup(cb_pad, zis.reshape(B * PIX), B * PIX)
    zqs = rows[:, :DIM].reshape(B, PIX, DIM).transpose(0, 2, 1)
    return (zis.reshape(B, 32, 32), zqs.reshape(B, DIM, 32, 32))


# speedup vs baseline: 1.6493x; 1.3022x over previous
import jax
import jax.numpy as jnp
from jax import lax
from jax.experimental import pallas as pl
from jax.experimental.pallas import tpu as pltpu

NUM_CODES = 1024
DIM = 64
PIX = 1024


def _vq_body(x_ref, cb_ref, zis_ref, zqs_ref, cbn2_ref, c2_ref, cbt_ref):
    @pl.when(pl.program_id(0) == 0)
    def _init():
        cb0 = cb_ref[...]
        cbn2_ref[...] = cb0 * -2.0
        c2_ref[...] = jnp.sum(cb0 * cb0, axis=1, keepdims=True)
        cbt_ref[...] = cb0.T

    for s in range(x_ref.shape[0]):
        x = x_ref[s]

        mmn2 = lax.dot_general(cbn2_ref[...], x, (((1,), (0,)), ((), ())),
                               precision=lax.Precision.DEFAULT)
        z2 = jnp.sum(x * x, axis=0)
        dist = (z2[None, :] + c2_ref[...]) + mmn2

        idx = jnp.argmin(dist, axis=0)
        ii = lax.broadcasted_iota(jnp.int32, (NUM_CODES, PIX), 0)
        zis_ref[s] = idx.reshape(8, 128)

        onehot = (ii == idx[None, :]).astype(jnp.float32)
        zq = lax.dot_general(cbt_ref[...], onehot, (((1,), (0,)), ((), ())),
                             precision=lax.Precision.DEFAULT)
        zqs_ref[s] = zq


BB = 2


def kernel(inputs, codebook):
    B = inputs.shape[0]
    x = inputs.reshape(B, DIM, PIX)
    zis, zqs = pl.pallas_call(
        _vq_body,
        grid=(B // BB,),
        in_specs=[
            pl.BlockSpec((BB, DIM, PIX), lambda b: (b, 0, 0)),
            pl.BlockSpec((NUM_CODES, DIM), lambda b: (0, 0)),
        ],
        out_specs=[
            pl.BlockSpec((BB, 8, 128), lambda b: (b, 0, 0)),
            pl.BlockSpec((BB, DIM, PIX), lambda b: (b, 0, 0)),
        ],
        out_shape=[
            jax.ShapeDtypeStruct((B, 8, 128), jnp.int32),
            jax.ShapeDtypeStruct((B, DIM, PIX), jnp.float32),
        ],
        scratch_shapes=[
            pltpu.VMEM((NUM_CODES, DIM), jnp.float32),
            pltpu.VMEM((NUM_CODES, 1), jnp.float32),
            pltpu.VMEM((DIM, NUM_CODES), jnp.float32),
        ],
    )(x, codebook)
    return (zis.reshape(B, 32, 32), zqs.reshape(B, DIM, 32, 32))


# speedup vs baseline: 1.6847x; 1.0215x over previous
import jax
import jax.numpy as jnp
from jax import lax
from jax.experimental import pallas as pl
from jax.experimental.pallas import tpu as pltpu

NUM_CODES = 1024
DIM = 64
PIX = 1024


def _vq_body(x_ref, cb_ref, zis_ref, zqs_ref, cbn2_ref, c2_ref, cbt_ref):
    @pl.when(pl.program_id(0) == 0)
    def _init():
        cb0 = cb_ref[...]
        cbn2_ref[...] = cb0 * -2.0
        c2_ref[...] = jnp.sum(cb0 * cb0, axis=1, keepdims=True)
        cbt_ref[...] = cb0.T

    for s in range(x_ref.shape[0]):
        x = x_ref[s]

        mmn2 = lax.dot_general(cbn2_ref[...], x, (((1,), (0,)), ((), ())),
                               precision=lax.Precision.DEFAULT)
        z2 = jnp.sum(x * x, axis=0)
        dist = (z2[None, :] + c2_ref[...]) + mmn2

        idx = jnp.argmin(dist, axis=0)
        ii = lax.broadcasted_iota(jnp.int32, (NUM_CODES, PIX), 0)
        zis_ref[s] = idx.reshape(8, 128)

        onehot = (ii == idx[None, :]).astype(jnp.float32)
        zq = lax.dot_general(cbt_ref[...], onehot, (((1,), (0,)), ((), ())),
                             precision=lax.Precision.DEFAULT)
        zqs_ref[s] = zq


BB = 4


def kernel(inputs, codebook):
    B = inputs.shape[0]
    x = inputs.reshape(B, DIM, PIX)
    zis, zqs = pl.pallas_call(
        _vq_body,
        grid=(B // BB,),
        in_specs=[
            pl.BlockSpec((BB, DIM, PIX), lambda b: (b, 0, 0)),
            pl.BlockSpec((NUM_CODES, DIM), lambda b: (0, 0)),
        ],
        out_specs=[
            pl.BlockSpec((BB, 8, 128), lambda b: (b, 0, 0)),
            pl.BlockSpec((BB, DIM, PIX), lambda b: (b, 0, 0)),
        ],
        out_shape=[
            jax.ShapeDtypeStruct((B, 8, 128), jnp.int32),
            jax.ShapeDtypeStruct((B, DIM, PIX), jnp.float32),
        ],
        scratch_shapes=[
            pltpu.VMEM((NUM_CODES, DIM), jnp.float32),
            pltpu.VMEM((NUM_CODES, 1), jnp.float32),
            pltpu.VMEM((DIM, NUM_CODES), jnp.float32),
        ],
    )(x, codebook)
    return (zis.reshape(B, 32, 32), zqs.reshape(B, DIM, 32, 32))
